# Initial kernel scaffold; baseline (speedup 1.0000x reference)
import jax
import jax.numpy as jnp
from jax.experimental import pallas as pl


def kernel(x, W, b):
    raise NotImplementedError("write your pallas kernel here")


# speedup vs baseline: 636.0905x; 636.0905x over previous
import functools

import jax
import jax.numpy as jnp
from jax.experimental import pallas as pl
from jax.experimental.pallas import tpu as pltpu

_BLK = 1024


def _moe_body(x_ref, w_ref, b_ref, o_ref):
    xb = x_ref[...]
    logits = (
        jnp.dot(xb, w_ref[...], preferred_element_type=jnp.float32)
        + b_ref[...]
    )
    n_exp = logits.shape[-1]
    neg_inf = jnp.float32(-jnp.inf)
    m1 = jnp.full(logits.shape[:-1] + (1,), neg_inf, jnp.float32)
    m2 = m1
    for e in range(n_exp):
        v = logits[:, e : e + 1]
        m2 = jnp.maximum(m2, jnp.minimum(m1, v))
        m1 = jnp.maximum(m1, v)
    t = jnp.exp(m2 - m1)
    s = 1.0 + t
    scale = 1.0 / s + t / s
    o_ref[...] = xb * scale


def kernel(x, W, b):
    B, S, E = x.shape
    n_exp = W.shape[1]
    tokens = B * S
    x2 = x.reshape(tokens, E)
    b2 = b.reshape(1, n_exp)
    grid = tokens // _BLK
    out = pl.pallas_call(
        _moe_body,
        grid=(grid,),
        in_specs=[
            pl.BlockSpec((_BLK, E), lambda i: (i, 0)),
            pl.BlockSpec((E, n_exp), lambda i: (0, 0)),
            pl.BlockSpec((1, n_exp), lambda i: (0, 0)),
        ],
        out_specs=pl.BlockSpec((_BLK, E), lambda i: (i, 0)),
        out_shape=jax.ShapeDtypeStruct((tokens, E), jnp.float32),
        compiler_params=pltpu.CompilerParams(
            dimension_semantics=("arbitrary",),
        ),
    )(x2, W, b2)
    return out.reshape(B, S, E)
